# baseline (speedup 1.0000x reference)
import functools

import jax
import jax.numpy as jnp
import numpy as np
from jax import lax
from jax.experimental import pallas as pl
from jax.experimental.pallas import tpu as pltpu
from jax.experimental.pallas import tpu_sc as plsc

F32 = jnp.float32
BF16 = jnp.bfloat16

_PERM = np.concatenate(
    [np.concatenate([g * 32 + 2 * np.arange(16),
                     g * 32 + 2 * np.arange(16) + 1]) for g in range(4)])

D = 128
DW = 64
ENT = 10000
NREL = 500
NHYP = 80000
NEDGE = 160000
NNODE = ENT + 6
NPAD = 10112
TAB = 512


def _tc0_body(pos_ref, walle_ref, waa_ref, rpad_ref, wrel_ref,
              mcat_ref, hta_ref, outr_ref):
    waa_h = waa_ref[...] * 0.5
    mcat_ref[...] = jnp.dot(pos_ref[...] * walle_ref[...], waa_h,
                            preferred_element_type=F32)
    hta_ref[...] = jnp.dot(rpad_ref[...], waa_h,
                           preferred_element_type=F32).astype(BF16)
    outr_ref[...] = jnp.dot(rpad_ref[...], wrel_ref[...],
                            preferred_element_type=F32)


_tc0 = pl.pallas_call(
    _tc0_body,
    out_shape=(jax.ShapeDtypeStruct((6 * D, D), F32),
               jax.ShapeDtypeStruct((TAB, D), BF16),
               jax.ShapeDtypeStruct((TAB, D), F32)),
)


def _tc1_body(e_ref, m_ref, g_ref):
    g_ref[...] = jnp.dot(e_ref[...], m_ref[...],
                         preferred_element_type=F32).astype(BF16)


_tc1 = pl.pallas_call(
    _tc1_body,
    grid=(6,),
    in_specs=[pl.BlockSpec((TAB, D), lambda k: (0, 0)),
              pl.BlockSpec((D, D), lambda k: (k, 0))],
    out_specs=pl.BlockSpec((TAB, D), lambda k: (k, 0)),
    out_shape=jax.ShapeDtypeStruct((6 * TAB, D), BF16),
)


def _tc2_body(s12_ref, e_ref, wa_ref, wb_ref, g_ref, b_ref, out_ref):
    x = jnp.dot(s12_ref[0], wa_ref[...], preferred_element_type=F32)
    x = x + jnp.dot(s12_ref[1], wb_ref[...], preferred_element_type=F32)
    x = 0.5 * x + 0.5 * e_ref[...]
    inv_n = 1.0 / float(NNODE)
    mean = jnp.sum(x, axis=0, keepdims=True) * inv_n
    var = jnp.sum(x * x, axis=0, keepdims=True) * inv_n - mean * mean
    inv = lax.rsqrt(var + 1e-5)
    out_ref[...] = jnp.tanh((x - mean) * (inv * g_ref[...]) + b_ref[...])


_tc2 = pl.pallas_call(
    _tc2_body,
    out_shape=jax.ShapeDtypeStruct((NPAD, D), F32),
)


_MESH = plsc.VectorSubcoreMesh(core_axis_name="c", subcore_axis_name="s")

CH = 64
NCHUNK1 = NHYP // CH
STEPS1 = 40


EROWS = NPAD // 32


@functools.partial(
    pl.kernel,
    out_type=jax.ShapeDtypeStruct((NHYP + NPAD, DW), jnp.int32),
    mesh=_MESH,
    compiler_params=pltpu.CompilerParams(use_tc_tiling_on_sc=False),
    scratch_types=(
        [pltpu.VMEM((7, CH), jnp.int32) for _ in range(2)]
        + [pltpu.VMEM((CH, DW), jnp.int32) for _ in range(14)]
        + [pltpu.SemaphoreType.DMA, pltpu.SemaphoreType.DMA]
    ),
)
def _sck1(hec_hbm, ut_hbm, ew_hbm, out_hbm, *refs):
    idxs = refs[0:2]
    bufs = (refs[2:9], refs[9:16])
    sems = refs[16:18]
    cid = lax.axis_index("c")
    sid = lax.axis_index("s")
    wid = sid * 2 + cid

    eb = wid * EROWS
    pltpu.sync_copy(ew_hbm.at[pl.ds(eb, EROWS)],
                    out_hbm.at[pl.ds(NHYP + eb, EROWS)])

    def load_fire(s, c):
        pltpu.sync_copy(hec_hbm.at[c], idxs[s])
        for k in range(1, 7):
            off = k * TAB
            for j in range(CH // 16):
                sl = pl.ds(j * 16, 16)
                idxs[s][k, sl] = idxs[s][k, sl] + off
        for k in range(7):
            pltpu.async_copy(ut_hbm.at[idxs[s].at[k]], bufs[s][k], sems[s])

    def wait_consume(s, c):
        for k in range(7):
            pltpu.make_async_copy(ut_hbm.at[idxs[s].at[k]], bufs[s][k],
                                  sems[s]).wait()
        b0, b1, b2, b3, b4, b5, b6 = bufs[s]

        def row(r, rc):
            for j in range(DW // 16):
                ix = (r, pl.ds(j * 16, 16))
                vs = [b[ix] for b in bufs[s]]
                es = [lax.bitcast_convert_type(v << 16, F32) for v in vs]
                os_ = [lax.bitcast_convert_type(v & -65536, F32) for v in vs]
                se = ((es[0] + es[1]) + (es[2] + es[3])
                      + ((es[4] + es[5]) + es[6]))
                so = ((os_[0] + os_[1]) + (os_[2] + os_[3])
                      + ((os_[4] + os_[5]) + os_[6]))
                ei = lax.bitcast_convert_type(se, jnp.int32) + 32768
                oi = lax.bitcast_convert_type(so, jnp.int32) + 32768
                b0[ix] = lax.shift_right_logical(ei, 16) | (oi & -65536)
            return rc

        lax.fori_loop(0, CH, row, 0)
        pltpu.sync_copy(b0, out_hbm.at[pl.ds(c * CH, CH)])

    def guarded(fn, s, c):
        @pl.when(c < NCHUNK1)
        def _():
            fn(s, c)

    guarded(load_fire, 0, wid)

    def step(u, carry):
        t0 = u * 2
        c0 = t0 * 32 + wid
        c1 = c0 + 32
        c2 = c0 + 64
        guarded(load_fire, 1, c1)
        guarded(wait_consume, 0, c0)
        guarded(load_fire, 0, c2)
        guarded(wait_consume, 1, c1)
        return carry

    lax.fori_loop(0, STEPS1 // 2, step, 0)


ECH = 128
NCHUNK2 = NEDGE // ECH
STEPS2 = 80
RPT = NPAD // 16


@functools.partial(
    pl.kernel,
    out_type=jax.ShapeDtypeStruct((2, NPAD, D), F32),
    mesh=_MESH,
    compiler_params=pltpu.CompilerParams(use_tc_tiling_on_sc=False),
    scratch_types=(
        pltpu.VMEM_SHARED((NPAD, D), F32),
        pltpu.VMEM((3, ECH), jnp.int32),
        pltpu.VMEM((3, ECH), jnp.int32),
        pltpu.VMEM((ECH, DW), jnp.int32),
        pltpu.VMEM((ECH, DW), jnp.int32),
        pltpu.VMEM((ECH, D), F32),
        pltpu.VMEM((32, D), F32),
        pltpu.SemaphoreType.DMA,
        pltpu.SemaphoreType.DMA,
    ),
)
def _sck2(ae_hbm, tab_hbm, s12_hbm,
          acc_sh, idx0, idx1, bbuf0, bbuf1, fbuf, zbuf, sem0, sem1):
    cid = lax.axis_index("c")
    sid = lax.axis_index("s")
    idxs = (idx0, idx1)
    bufs = (bbuf0, bbuf1)
    sems = (sem0, sem1)

    def zrow(r, carry):
        for j in range(D // 16):
            zbuf[r, pl.ds(j * 16, 16)] = jnp.zeros((16,), F32)
        return carry

    lax.fori_loop(0, 32, zrow, 0)
    rbase = sid * RPT
    for q in range(RPT // 32):
        pltpu.sync_copy(zbuf, acc_sh.at[pl.ds(rbase + q * 32, 32)])
    rem = RPT % 32
    if rem:
        pltpu.sync_copy(zbuf.at[pl.ds(0, rem)],
                        acc_sh.at[pl.ds(rbase + RPT - rem, rem)])
    plsc.subcore_barrier()

    def load_fire(s, c):
        pltpu.sync_copy(ae_hbm.at[c], idxs[s])
        pltpu.async_copy(tab_hbm.at[idxs[s].at[cid]], bufs[s], sems[s])

    def wait_scatter(s, c):
        pltpu.make_async_copy(tab_hbm.at[idxs[s].at[cid]], bufs[s],
                              sems[s]).wait()
        bb = bufs[s]
        fb = fbuf

        def row(r, rc):
            for g in range(DW // 16):
                v = bb[r, pl.ds(g * 16, 16)]
                fb[r, pl.ds(g * 32, 16)] = lax.bitcast_convert_type(
                    v << 16, F32)
                fb[r, pl.ds(g * 32 + 16, 16)] = lax.bitcast_convert_type(
                    v & -65536, F32)
            return rc

        lax.fori_loop(0, ECH, row, 0)
        pltpu.sync_copy(fb, acc_sh.at[idxs[s].at[2]], add=True)

    def guarded(fn, s, c):
        @pl.when(c < NCHUNK2)
        def _():
            fn(s, c)

    guarded(load_fire, 0, sid)

    def step(u, carry):
        c0 = u * 32 + sid
        c1 = c0 + 16
        c2 = c0 + 32
        guarded(load_fire, 1, c1)
        guarded(wait_scatter, 0, c0)
        guarded(load_fire, 0, c2)
        guarded(wait_scatter, 1, c1)
        return carry

    lax.fori_loop(0, STEPS2 // 2, step, 0)
    plsc.subcore_barrier()
    pltpu.sync_copy(acc_sh.at[pl.ds(rbase, RPT)],
                    s12_hbm.at[cid, pl.ds(rbase, RPT)])


def kernel(hyperedge, edge_index, edge_type, E_in, R_in, w_alle, w_addpos,
           w_alleandr, w_rel, bn_gamma, bn_beta):
    e512 = E_in[:TAB]
    pos = E_in[ENT:ENT + 6].reshape(6 * D, 1)
    r_pad = jnp.pad(R_in, ((0, TAB - NREL), (0, 0)))
    e_pad = jnp.pad(E_in, ((0, NPAD - NNODE), (0, 0)))
    hec = hyperedge.T.reshape(7, NCHUNK1, CH).transpose(1, 0, 2)
    dst = edge_index[0]
    src = edge_index[1]
    ae = jnp.stack([edge_type.reshape(NCHUNK2, ECH),
                    src.reshape(NCHUNK2, ECH) + NHYP,
                    dst.reshape(NCHUNK2, ECH)], axis=1)

    mcat, hta, outr = _tc0(pos, w_alle, w_alleandr, r_pad, w_rel)
    gtab = _tc1(e512, mcat)
    ut = jnp.concatenate([hta, gtab], axis=0)
    ut_w = lax.bitcast_convert_type(ut.reshape(7 * TAB, DW, 2), jnp.int32)
    e_w = lax.bitcast_convert_type(
        e_pad.astype(BF16).reshape(NPAD, DW, 2), jnp.int32)
    tab = _sck1(hec, ut_w, e_w)
    s12 = _sck2(ae, tab)
    out = _tc2(s12, e_pad, w_addpos[:D][_PERM], w_addpos[D:][_PERM],
               bn_gamma.reshape(1, D), bn_beta.reshape(1, D))
    return (out[:NNODE], outr[:NREL])


# speedup vs baseline: 1.6640x; 1.6640x over previous
import functools

import jax
import jax.numpy as jnp
from jax import lax
from jax.experimental import pallas as pl
from jax.experimental.pallas import tpu as pltpu
from jax.experimental.pallas import tpu_sc as plsc

F32 = jnp.float32
D = 128
ENT = 10000
NREL = 500
NHYP = 80000
NEDGE = 160000
NNODE = ENT + 6
NPAD = 10112
TAB = 512


def _tc0_body(pos_ref, walle_ref, waa_ref, rpad_ref, wrel_ref,
              mcat_ref, hta_ref, outr_ref):
    waa_h = waa_ref[...] * 0.5
    mcat_ref[...] = jnp.dot(pos_ref[...] * walle_ref[...], waa_h,
                            preferred_element_type=F32)
    hta_ref[...] = jnp.dot(rpad_ref[...], waa_h, preferred_element_type=F32)
    outr_ref[...] = jnp.dot(rpad_ref[...], wrel_ref[...],
                            preferred_element_type=F32)


_tc0 = pl.pallas_call(
    _tc0_body,
    out_shape=(jax.ShapeDtypeStruct((6 * D, D), F32),
               jax.ShapeDtypeStruct((TAB, D), F32),
               jax.ShapeDtypeStruct((TAB, D), F32)),
)


def _tc1_body(e_ref, m_ref, g_ref):
    g_ref[...] = jnp.dot(e_ref[...], m_ref[...], preferred_element_type=F32)


_tc1 = pl.pallas_call(
    _tc1_body,
    grid=(6,),
    in_specs=[pl.BlockSpec((TAB, D), lambda k: (0, 0)),
              pl.BlockSpec((D, D), lambda k: (k, 0))],
    out_specs=pl.BlockSpec((TAB, D), lambda k: (k, 0)),
    out_shape=jax.ShapeDtypeStruct((6 * TAB, D), F32),
)


def _tc2_body(s1_ref, s2_ref, e_ref, wa_ref, wb_ref, g_ref, b_ref, out_ref):
    x = jnp.dot(s1_ref[...], wa_ref[...], preferred_element_type=F32)
    x = x + jnp.dot(s2_ref[...], wb_ref[...], preferred_element_type=F32)
    x = 0.5 * x + 0.5 * e_ref[...]
    inv_n = 1.0 / float(NNODE)
    mean = jnp.sum(x, axis=0, keepdims=True) * inv_n
    var = jnp.sum(x * x, axis=0, keepdims=True) * inv_n - mean * mean
    inv = lax.rsqrt(var + 1e-5)
    out_ref[...] = jnp.tanh((x - mean) * (inv * g_ref[...]) + b_ref[...])


_tc2 = pl.pallas_call(
    _tc2_body,
    out_shape=jax.ShapeDtypeStruct((NPAD, D), F32),
)


_MESH = plsc.VectorSubcoreMesh(core_axis_name="c", subcore_axis_name="s")

CH = 64
NCHUNK1 = NHYP // CH
STEPS1 = 40


@functools.partial(
    pl.kernel,
    out_type=jax.ShapeDtypeStruct((NHYP, D), F32),
    mesh=_MESH,
    scratch_types=(
        [pltpu.VMEM((7, CH), jnp.int32) for _ in range(2)]
        + [pltpu.VMEM((CH, D), F32) for _ in range(14)]
        + [pltpu.SemaphoreType.DMA, pltpu.SemaphoreType.DMA]
    ),
)
def _sck1(hec_hbm, ut_hbm, out_hbm, *refs):
    idxs = refs[0:2]
    bufs = (refs[2:9], refs[9:16])
    sems = refs[16:18]
    cid = lax.axis_index("c")
    sid = lax.axis_index("s")
    wid = sid * 2 + cid

    def load_fire(s, c):
        pltpu.sync_copy(hec_hbm.at[c], idxs[s])
        for k in range(1, 7):
            off = k * TAB
            for j in range(CH // 16):
                sl = pl.ds(j * 16, 16)
                idxs[s][k, sl] = idxs[s][k, sl] + off
        for k in range(7):
            pltpu.async_copy(ut_hbm.at[idxs[s].at[k]], bufs[s][k], sems[s])

    def wait_consume(s, c):
        for k in range(7):
            pltpu.make_async_copy(ut_hbm.at[idxs[s].at[k]], bufs[s][k],
                                  sems[s]).wait()
        b0, b1, b2, b3, b4, b5, b6 = bufs[s]

        def row(r, rc):
            for j in range(D // 16):
                ix = (r, pl.ds(j * 16, 16))
                b0[ix] = ((b0[ix] + b1[ix]) + (b2[ix] + b3[ix])
                          + ((b4[ix] + b5[ix]) + b6[ix]))
            return rc

        lax.fori_loop(0, CH, row, 0)
        pltpu.sync_copy(b0, out_hbm.at[pl.ds(c * CH, CH)])

    def guarded(fn, s, c):
        @pl.when(c < NCHUNK1)
        def _():
            fn(s, c)

    guarded(load_fire, 0, wid)

    def step(u, carry):
        t0 = u * 2
        c0 = t0 * 32 + wid
        c1 = c0 + 32
        c2 = c0 + 64
        guarded(load_fire, 1, c1)
        guarded(wait_consume, 0, c0)
        guarded(load_fire, 0, c2)
        guarded(wait_consume, 1, c1)
        return carry

    lax.fori_loop(0, STEPS1 // 2, step, 0)


ECH = 128
NCHUNK2 = NEDGE // ECH
STEPS2 = 80
RPT = NPAD // 16


@functools.partial(
    pl.kernel,
    out_type=(jax.ShapeDtypeStruct((NPAD, D), F32),
              jax.ShapeDtypeStruct((NPAD, D), F32)),
    mesh=_MESH,
    scratch_types=(
        pltpu.VMEM_SHARED((NPAD, D), F32),
        pltpu.VMEM((3, ECH), jnp.int32),
        pltpu.VMEM((3, ECH), jnp.int32),
        pltpu.VMEM((ECH, D), F32),
        pltpu.VMEM((ECH, D), F32),
        pltpu.VMEM((64, D), F32),
        pltpu.SemaphoreType.DMA,
        pltpu.SemaphoreType.DMA,
    ),
)
def _sck2(ae_hbm, hemb_hbm, epad_hbm, s1_hbm, s2_hbm,
          acc_sh, idx0, idx1, buf0, buf1, zbuf, sem0, sem1):
    cid = lax.axis_index("c")
    sid = lax.axis_index("s")
    idxs = (idx0, idx1)
    bufs = (buf0, buf1)
    sems = (sem0, sem1)

    def zrow(r, carry):
        for j in range(D // 16):
            zbuf[r, pl.ds(j * 16, 16)] = jnp.zeros((16,), F32)
        return carry

    lax.fori_loop(0, 64, zrow, 0)
    rbase = sid * RPT
    for q in range(RPT // 64):
        pltpu.sync_copy(zbuf, acc_sh.at[pl.ds(rbase + q * 64, 64)])
    rem = RPT % 64
    if rem:
        pltpu.sync_copy(zbuf.at[pl.ds(0, rem)],
                        acc_sh.at[pl.ds(rbase + RPT - rem, rem)])
    plsc.subcore_barrier()

    def load_fire(s, c):
        pltpu.sync_copy(ae_hbm.at[c], idxs[s])

        @pl.when(cid == 0)
        def _():
            pltpu.async_copy(hemb_hbm.at[idxs[s].at[0]], bufs[s], sems[s])

        @pl.when(cid == 1)
        def _():
            pltpu.async_copy(epad_hbm.at[idxs[s].at[1]], bufs[s], sems[s])

    def wait_scatter(s, c):
        @pl.when(cid == 0)
        def _():
            pltpu.make_async_copy(hemb_hbm.at[idxs[s].at[0]], bufs[s],
                                  sems[s]).wait()

        @pl.when(cid == 1)
        def _():
            pltpu.make_async_copy(epad_hbm.at[idxs[s].at[1]], bufs[s],
                                  sems[s]).wait()

        pltpu.sync_copy(bufs[s], acc_sh.at[idxs[s].at[2]], add=True)

    def guarded(fn, s, c):
        @pl.when(c < NCHUNK2)
        def _():
            fn(s, c)

    guarded(load_fire, 0, sid)

    def step(u, carry):
        c0 = u * 32 + sid
        c1 = c0 + 16
        c2 = c0 + 32
        guarded(load_fire, 1, c1)
        guarded(wait_scatter, 0, c0)
        guarded(load_fire, 0, c2)
        guarded(wait_scatter, 1, c1)
        return carry

    lax.fori_loop(0, STEPS2 // 2, step, 0)
    plsc.subcore_barrier()

    @pl.when(cid == 0)
    def _():
        pltpu.sync_copy(acc_sh.at[pl.ds(rbase, RPT)],
                        s1_hbm.at[pl.ds(rbase, RPT)])

    @pl.when(cid == 1)
    def _():
        pltpu.sync_copy(acc_sh.at[pl.ds(rbase, RPT)],
                        s2_hbm.at[pl.ds(rbase, RPT)])


def kernel(hyperedge, edge_index, edge_type, E_in, R_in, w_alle, w_addpos,
           w_alleandr, w_rel, bn_gamma, bn_beta):
    e512 = E_in[:TAB]
    pos = E_in[ENT:ENT + 6].reshape(6 * D, 1)
    r_pad = jnp.pad(R_in, ((0, TAB - NREL), (0, 0)))
    e_pad = jnp.pad(E_in, ((0, NPAD - NNODE), (0, 0)))
    hec = hyperedge.T.reshape(7, NCHUNK1, CH).transpose(1, 0, 2)
    dst = edge_index[0]
    src = edge_index[1]
    ae = jnp.stack([edge_type.reshape(NCHUNK2, ECH),
                    src.reshape(NCHUNK2, ECH),
                    dst.reshape(NCHUNK2, ECH)], axis=1)

    mcat, hta, outr = _tc0(pos, w_alle, w_alleandr, r_pad, w_rel)
    gtab = _tc1(e512, mcat)
    ut = jnp.concatenate([hta, gtab], axis=0)
    hemb = _sck1(hec, ut)
    s1, s2 = _sck2(ae, hemb, e_pad)
    out = _tc2(s1, s2, e_pad, w_addpos[:D], w_addpos[D:],
               bn_gamma.reshape(1, D), bn_beta.reshape(1, D))
    return (out[:NNODE], outr[:NREL])


# speedup vs baseline: 1.6796x; 1.0094x over previous
import functools

import jax
import jax.numpy as jnp
from jax import lax
from jax.experimental import pallas as pl
from jax.experimental.pallas import tpu as pltpu
from jax.experimental.pallas import tpu_sc as plsc

F32 = jnp.float32
D = 128
ENT = 10000
NREL = 500
NHYP = 80000
NEDGE = 160000
NNODE = ENT + 6
NPAD = 10112
TAB = 512


def _tc1_body(e_ref, rpad_ref, pos_ref, walle_ref, waa_ref, wrel_ref,
              ut_ref, outr_ref):
    i = pl.program_id(0)
    waa_h = waa_ref[...] * 0.5

    @pl.when(i == 0)
    def _():
        ut_ref[...] = jnp.dot(rpad_ref[...], waa_h,
                              preferred_element_type=F32)
        outr_ref[...] = jnp.dot(rpad_ref[...], wrel_ref[...],
                                preferred_element_type=F32)

    @pl.when(i > 0)
    def _():
        m = jnp.dot(pos_ref[...] * walle_ref[...], waa_h,
                    preferred_element_type=F32)
        ut_ref[...] = jnp.dot(e_ref[...], m, preferred_element_type=F32)


_tc1 = pl.pallas_call(
    _tc1_body,
    grid=(7,),
    in_specs=[pl.BlockSpec((TAB, D), lambda k: (0, 0)),
              pl.BlockSpec((TAB, D), lambda k: (0, 0)),
              pl.BlockSpec((D, 1), lambda k: (jnp.maximum(k - 1, 0), 0)),
              pl.BlockSpec((D, D), lambda k: (jnp.maximum(k - 1, 0), 0)),
              pl.BlockSpec((D, D), lambda k: (0, 0)),
              pl.BlockSpec((D, D), lambda k: (0, 0))],
    out_specs=(pl.BlockSpec((TAB, D), lambda k: (k, 0)),
               pl.BlockSpec((TAB, D), lambda k: (0, 0))),
    out_shape=(jax.ShapeDtypeStruct((7 * TAB, D), F32),
               jax.ShapeDtypeStruct((TAB, D), F32)),
)


def _tc2_body(s1_ref, s2_ref, e_ref, wa_ref, wb_ref, g_ref, b_ref, out_ref):
    x = jnp.dot(s1_ref[...], wa_ref[...], preferred_element_type=F32)
    x = x + jnp.dot(s2_ref[...], wb_ref[...], preferred_element_type=F32)
    x = 0.5 * x + 0.5 * e_ref[...]
    inv_n = 1.0 / float(NNODE)
    mean = jnp.sum(x, axis=0, keepdims=True) * inv_n
    var = jnp.sum(x * x, axis=0, keepdims=True) * inv_n - mean * mean
    inv = lax.rsqrt(var + 1e-5)
    out_ref[...] = jnp.tanh((x - mean) * (inv * g_ref[...]) + b_ref[...])


_tc2 = pl.pallas_call(
    _tc2_body,
    out_shape=jax.ShapeDtypeStruct((NPAD, D), F32),
)


_MESH = plsc.VectorSubcoreMesh(core_axis_name="c", subcore_axis_name="s")

CH = 64
NCHUNK1 = NHYP // CH
STEPS1 = 40


@functools.partial(
    pl.kernel,
    out_type=jax.ShapeDtypeStruct((NHYP, D), F32),
    mesh=_MESH,
    scratch_types=(
        [pltpu.VMEM((7, CH), jnp.int32) for _ in range(2)]
        + [pltpu.VMEM((CH, D), F32) for _ in range(14)]
        + [pltpu.SemaphoreType.DMA, pltpu.SemaphoreType.DMA]
    ),
)
def _sck1(hec_hbm, ut_hbm, out_hbm, *refs):
    idxs = refs[0:2]
    bufs = (refs[2:9], refs[9:16])
    sems = refs[16:18]
    cid = lax.axis_index("c")
    sid = lax.axis_index("s")
    wid = sid * 2 + cid

    def load_fire(s, c):
        pltpu.sync_copy(hec_hbm.at[c], idxs[s])
        for k in range(1, 7):
            off = k * TAB
            for j in range(CH // 16):
                sl = pl.ds(j * 16, 16)
                idxs[s][k, sl] = idxs[s][k, sl] + off
        for k in range(7):
            pltpu.async_copy(ut_hbm.at[idxs[s].at[k]], bufs[s][k], sems[s])

    def wait_consume(s, c):
        for k in range(7):
            pltpu.make_async_copy(ut_hbm.at[idxs[s].at[k]], bufs[s][k],
                                  sems[s]).wait()
        b0, b1, b2, b3, b4, b5, b6 = bufs[s]

        def row(r, rc):
            for j in range(D // 16):
                ix = (r, pl.ds(j * 16, 16))
                b0[ix] = ((b0[ix] + b1[ix]) + (b2[ix] + b3[ix])
                          + ((b4[ix] + b5[ix]) + b6[ix]))
            return rc

        lax.fori_loop(0, CH, row, 0)
        pltpu.sync_copy(b0, out_hbm.at[pl.ds(c * CH, CH)])

    def guarded(fn, s, c):
        @pl.when(c < NCHUNK1)
        def _():
            fn(s, c)

    guarded(load_fire, 0, wid)

    def step(u, carry):
        t0 = u * 2
        c0 = t0 * 32 + wid
        c1 = c0 + 32
        c2 = c0 + 64
        guarded(load_fire, 1, c1)
        guarded(wait_consume, 0, c0)
        guarded(load_fire, 0, c2)
        guarded(wait_consume, 1, c1)
        return carry

    lax.fori_loop(0, STEPS1 // 2, step, 0)


ECH = 128
NCHUNK2 = NEDGE // ECH
STEPS2 = 80
RPT = NPAD // 16


@functools.partial(
    pl.kernel,
    out_type=(jax.ShapeDtypeStruct((NPAD, D), F32),
              jax.ShapeDtypeStruct((NPAD, D), F32)),
    mesh=_MESH,
    scratch_types=(
        pltpu.VMEM_SHARED((NPAD, D), F32),
        pltpu.VMEM((3, ECH), jnp.int32),
        pltpu.VMEM((3, ECH), jnp.int32),
        pltpu.VMEM((ECH, D), F32),
        pltpu.VMEM((ECH, D), F32),
        pltpu.VMEM((64, D), F32),
        pltpu.SemaphoreType.DMA,
        pltpu.SemaphoreType.DMA,
    ),
)
def _sck2(ae_hbm, hemb_hbm, epad_hbm, s1_hbm, s2_hbm,
          acc_sh, idx0, idx1, buf0, buf1, zbuf, sem0, sem1):
    cid = lax.axis_index("c")
    sid = lax.axis_index("s")
    idxs = (idx0, idx1)
    bufs = (buf0, buf1)
    sems = (sem0, sem1)

    def zrow(r, carry):
        for j in range(D // 16):
            zbuf[r, pl.ds(j * 16, 16)] = jnp.zeros((16,), F32)
        return carry

    lax.fori_loop(0, 64, zrow, 0)
    rbase = sid * RPT
    for q in range(RPT // 64):
        pltpu.sync_copy(zbuf, acc_sh.at[pl.ds(rbase + q * 64, 64)])
    rem = RPT % 64
    if rem:
        pltpu.sync_copy(zbuf.at[pl.ds(0, rem)],
                        acc_sh.at[pl.ds(rbase + RPT - rem, rem)])
    plsc.subcore_barrier()

    def load_fire(s, c):
        pltpu.sync_copy(ae_hbm.at[c], idxs[s])

        @pl.when(cid == 0)
        def _():
            pltpu.async_copy(hemb_hbm.at[idxs[s].at[0]], bufs[s], sems[s])

        @pl.when(cid == 1)
        def _():
            pltpu.async_copy(epad_hbm.at[idxs[s].at[1]], bufs[s], sems[s])

    def wait_scatter(s, c):
        @pl.when(cid == 0)
        def _():
            pltpu.make_async_copy(hemb_hbm.at[idxs[s].at[0]], bufs[s],
                                  sems[s]).wait()

        @pl.when(cid == 1)
        def _():
            pltpu.make_async_copy(epad_hbm.at[idxs[s].at[1]], bufs[s],
                                  sems[s]).wait()

        pltpu.sync_copy(bufs[s], acc_sh.at[idxs[s].at[2]], add=True)

    def guarded(fn, s, c):
        @pl.when(c < NCHUNK2)
        def _():
            fn(s, c)

    guarded(load_fire, 0, sid)

    def step(u, carry):
        c0 = u * 32 + sid
        c1 = c0 + 16
        c2 = c0 + 32
        guarded(load_fire, 1, c1)
        guarded(wait_scatter, 0, c0)
        guarded(load_fire, 0, c2)
        guarded(wait_scatter, 1, c1)
        return carry

    lax.fori_loop(0, STEPS2 // 2, step, 0)
    plsc.subcore_barrier()

    @pl.when(cid == 0)
    def _():
        pltpu.sync_copy(acc_sh.at[pl.ds(rbase, RPT)],
                        s1_hbm.at[pl.ds(rbase, RPT)])

    @pl.when(cid == 1)
    def _():
        pltpu.sync_copy(acc_sh.at[pl.ds(rbase, RPT)],
                        s2_hbm.at[pl.ds(rbase, RPT)])


def kernel(hyperedge, edge_index, edge_type, E_in, R_in, w_alle, w_addpos,
           w_alleandr, w_rel, bn_gamma, bn_beta):
    e512 = E_in[:TAB]
    pos = E_in[ENT:ENT + 6].reshape(6 * D, 1)
    r_pad = jnp.pad(R_in, ((0, TAB - NREL), (0, 0)))
    e_pad = jnp.pad(E_in, ((0, NPAD - NNODE), (0, 0)))
    hec = hyperedge.T.reshape(7, NCHUNK1, CH).transpose(1, 0, 2)
    dst = edge_index[0]
    src = edge_index[1]
    ae = jnp.stack([edge_type.reshape(NCHUNK2, ECH),
                    src.reshape(NCHUNK2, ECH),
                    dst.reshape(NCHUNK2, ECH)], axis=1)

    ut, outr = _tc1(e512, r_pad, pos, w_alle, w_alleandr, w_rel)
    hemb = _sck1(hec, ut)
    s1, s2 = _sck2(ae, hemb, e_pad)
    out = _tc2(s1, s2, e_pad, w_addpos[:D], w_addpos[D:],
               bn_gamma.reshape(1, D), bn_beta.reshape(1, D))
    return (out[:NNODE], outr[:NREL])
